# baseline (speedup 1.0000x reference)
import functools

import jax
import jax.numpy as jnp
from jax import lax
from jax.experimental import pallas as pl
from jax.experimental.pallas import tpu as pltpu
from jax.experimental.pallas import tpu_sc as plsc

N_FIELDS = 26
DIM = 32
BATCH = 16384
VOCAB_SIZE = 100000
CHUNK = 4096
NCHUNKS = BATCH // CHUNK


def kernel(inputs, tables):
    idx_t = inputs.T
    tab_t = tables.transpose(0, 2, 1)

    mesh = plsc.VectorSubcoreMesh(core_axis_name="c", subcore_axis_name="s")
    assert mesh.num_cores * mesh.num_subcores == DIM

    @functools.partial(
        pl.kernel,
        out_type=tuple(
            jax.ShapeDtypeStruct((DIM, BATCH), jnp.float32)
            for _ in range(N_FIELDS)
        ),
        mesh=mesh,
        compiler_params=pltpu.CompilerParams(needs_layout_passes=False),
        scratch_types=[
            pltpu.VMEM((VOCAB_SIZE,), jnp.float32),
            pltpu.VMEM((2, CHUNK), jnp.int32),
            pltpu.VMEM((2, CHUNK), jnp.float32),
            pltpu.VMEM_SHARED((8, BATCH), jnp.int32),
            pltpu.SemaphoreType.DMA,
            pltpu.SemaphoreType.DMA,
            pltpu.SemaphoreType.DMA,
            pltpu.SemaphoreType.DMA,
        ],
    )
    def run(idx_hbm, tab_hbm, *rest):
        outs = rest[:N_FIELDS]
        (row_v, idx_v, out_v, sh_idx, shidx_sem, idx_sem, out_sem,
         row_sem) = rest[N_FIELDS:]
        cid = lax.axis_index("c")
        sid = lax.axis_index("s")
        wid = sid * mesh.num_cores + cid

        def idx_copy(f, c, b):
            return pltpu.make_async_copy(
                sh_idx.at[f % 8, pl.ds(c * CHUNK, CHUNK)], idx_v.at[b], idx_sem
            )

        def shidx_copy(f):
            return pltpu.make_async_copy(
                idx_hbm.at[f], sh_idx.at[f % 8], shidx_sem
            )

        def out_copy(f, c, b):
            return pltpu.make_async_copy(
                out_v.at[b],
                outs[f].at[wid, pl.ds(c * CHUNK, CHUNK)],
                out_sem,
            )

        @pl.when(sid == 0)
        def _():
            for g in range(4):
                shidx_copy(g).start()
            for g in range(4):
                shidx_copy(g).wait()
            for g in range(4, 8):
                shidx_copy(g).start()

        pltpu.sync_copy(tab_hbm.at[0, wid], row_v)
        plsc.subcore_barrier()
        idx_copy(0, 0, 0).start()

        for f in range(N_FIELDS):

            def chunk_body(c, _, f=f):
                b = c & 1
                idx_copy(f, c, b).wait()

                @pl.when(c < NCHUNKS - 1)
                def _():
                    idx_copy(f, c + 1, 1 - b).start()

                @pl.when(c >= 2)
                def _():
                    out_copy(f, c - 2, b).wait()

                @plsc.parallel_loop(0, CHUNK, 16, unroll=16)
                def _(i):
                    iv = idx_v[b, pl.ds(i, 16)]
                    out_v[b, pl.ds(i, 16)] = plsc.load_gather(row_v, [iv])

                out_copy(f, c, b).start()
                return 0

            lax.fori_loop(0, NCHUNKS, chunk_body, 0)

            if f % 4 == 3 and f + 1 < N_FIELDS:
                outstanding = min(f + 5, N_FIELDS) - (f + 1)

                @pl.when(sid == 0)
                def _(f=f, n=outstanding):
                    for k in range(n):
                        shidx_copy(f + 1 + k).wait()

                plsc.subcore_barrier()

                if f + 5 < N_FIELDS:

                    @pl.when(sid == 0)
                    def _(f=f):
                        for g in range(f + 5, min(f + 9, N_FIELDS)):
                            shidx_copy(g).start()

            if f + 1 < N_FIELDS:
                idx_copy(f + 1, 0, 0).start()
                row_cp = pltpu.make_async_copy(
                    tab_hbm.at[f + 1, wid], row_v, row_sem
                )
                row_cp.start()
                out_copy(f, NCHUNKS - 2, 0).wait()
                out_copy(f, NCHUNKS - 1, 1).wait()
                row_cp.wait()
            else:
                out_copy(f, NCHUNKS - 2, 0).wait()
                out_copy(f, NCHUNKS - 1, 1).wait()

    outs = run(idx_t, tab_t)
    return tuple(o.T for o in outs)


# speedup vs baseline: 1.0020x; 1.0020x over previous
import functools

import jax
import jax.numpy as jnp
from jax import lax
from jax.experimental import pallas as pl
from jax.experimental.pallas import tpu as pltpu
from jax.experimental.pallas import tpu_sc as plsc

N_FIELDS = 26
DIM = 32
BATCH = 16384
VOCAB_SIZE = 100000
CHUNK = 4096
NCHUNKS = BATCH // CHUNK


def kernel(inputs, tables):
    idx_t = inputs.T
    tab_t = tables.transpose(0, 2, 1)

    mesh = plsc.VectorSubcoreMesh(core_axis_name="c", subcore_axis_name="s")
    assert mesh.num_cores * mesh.num_subcores == DIM

    @functools.partial(
        pl.kernel,
        out_type=tuple(
            jax.ShapeDtypeStruct((DIM, BATCH), jnp.float32)
            for _ in range(N_FIELDS)
        ),
        mesh=mesh,
        compiler_params=pltpu.CompilerParams(needs_layout_passes=False),
        scratch_types=[
            pltpu.VMEM((VOCAB_SIZE,), jnp.float32),
            pltpu.VMEM((2, CHUNK), jnp.int32),
            pltpu.VMEM((2, CHUNK), jnp.float32),
            pltpu.VMEM_SHARED((8, BATCH), jnp.int32),
            pltpu.SemaphoreType.DMA,
            pltpu.SemaphoreType.DMA,
            pltpu.SemaphoreType.DMA,
            pltpu.SemaphoreType.DMA,
        ],
    )
    def run(idx_hbm, tab_hbm, *rest):
        outs = rest[:N_FIELDS]
        (row_v, idx_v, out_v, sh_idx, shidx_sem, idx_sem, out_sem,
         row_sem) = rest[N_FIELDS:]
        cid = lax.axis_index("c")
        sid = lax.axis_index("s")
        wid = sid * mesh.num_cores + cid

        def idx_copy(f, c, b):
            return pltpu.make_async_copy(
                sh_idx.at[f % 8, pl.ds(c * CHUNK, CHUNK)], idx_v.at[b], idx_sem
            )

        def shidx_copy(f):
            return pltpu.make_async_copy(
                idx_hbm.at[f], sh_idx.at[f % 8], shidx_sem
            )

        def out_copy(f, c, b):
            return pltpu.make_async_copy(
                out_v.at[b],
                outs[f].at[wid, pl.ds(c * CHUNK, CHUNK)],
                out_sem,
            )

        @pl.when(sid == 0)
        def _():
            for g in range(4):
                shidx_copy(g).start()
            for g in range(4):
                shidx_copy(g).wait()
            for g in range(4, 8):
                shidx_copy(g).start()

        pltpu.sync_copy(tab_hbm.at[0, wid], row_v)
        plsc.subcore_barrier()
        idx_copy(0, 0, 0).start()

        for f in range(N_FIELDS):

            def chunk_body(c, _, f=f):
                b = c & 1
                idx_copy(f, c, b).wait()

                @pl.when(c < NCHUNKS - 1)
                def _():
                    idx_copy(f, c + 1, 1 - b).start()

                @pl.when(c >= 2)
                def _():
                    out_copy(f, c - 2, b).wait()

                @plsc.parallel_loop(0, CHUNK, 16, unroll=8)
                def _(i):
                    iv = idx_v[b, pl.ds(i, 16)]
                    out_v[b, pl.ds(i, 16)] = plsc.load_gather(row_v, [iv])

                out_copy(f, c, b).start()
                return 0

            lax.fori_loop(0, NCHUNKS, chunk_body, 0)

            if f % 4 == 3 and f + 1 < N_FIELDS:
                outstanding = min(f + 5, N_FIELDS) - (f + 1)

                @pl.when(sid == 0)
                def _(f=f, n=outstanding):
                    for k in range(n):
                        shidx_copy(f + 1 + k).wait()

                plsc.subcore_barrier()

                if f + 5 < N_FIELDS:

                    @pl.when(sid == 0)
                    def _(f=f):
                        for g in range(f + 5, min(f + 9, N_FIELDS)):
                            shidx_copy(g).start()

            if f + 1 < N_FIELDS:
                idx_copy(f + 1, 0, 0).start()
                row_cp = pltpu.make_async_copy(
                    tab_hbm.at[f + 1, wid], row_v, row_sem
                )
                row_cp.start()
                out_copy(f, NCHUNKS - 2, 0).wait()
                out_copy(f, NCHUNKS - 1, 1).wait()
                row_cp.wait()
            else:
                out_copy(f, NCHUNKS - 2, 0).wait()
                out_copy(f, NCHUNKS - 1, 1).wait()

    outs = run(idx_t, tab_t)
    return tuple(o.T for o in outs)
